# Initial kernel scaffold; baseline (speedup 1.0000x reference)
import jax
import jax.numpy as jnp
from jax.experimental import pallas as pl


def kernel(x):
    raise NotImplementedError("write your pallas kernel here")


# speedup vs baseline: 8.3501x; 8.3501x over previous
import functools

import jax
import jax.numpy as jnp
from jax import lax
from jax.experimental import pallas as pl
from jax.experimental.pallas import tpu as pltpu
from jax.experimental.pallas import tpu_sc as plsc

B = 128
N = 32768
K = N // 8
L = 16
NCHUNK = N // L
NW = 32
ROWS_PER_W = B // NW

_MASK31 = 0x7FFFFFFF


def _mono(v):
    b = lax.bitcast_convert_type(v, jnp.int32)
    s = jnp.right_shift(b, 31)
    return jnp.bitwise_xor(b, jnp.bitwise_and(s, _MASK31))


def _find_bin(hist_ref, nbins, kt):
    chunks = nbins // L
    lane_iota = lax.iota(jnp.int32, L)

    def body(i, carry):
        acc, b_chunk, acc_above, chunk_sav = carry
        j = chunks - 1 - i
        chunk = hist_ref[pl.ds(j * L, L)]
        csum = jnp.sum(chunk)
        take = jnp.logical_and(b_chunk < 0, acc + csum >= kt)
        b_chunk = jnp.where(take, j, b_chunk).astype(jnp.int32)
        acc_above = jnp.where(take, acc, acc_above)
        takev = jnp.broadcast_to(take, (L,))
        chunk_sav = jnp.where(takev, chunk, chunk_sav)
        return acc + csum, b_chunk, acc_above, chunk_sav

    init = (jnp.int32(0), jnp.int32(-1), jnp.int32(0), jnp.zeros((L,), jnp.int32))
    acc, b_chunk, acc_above, chunk_sav = lax.fori_loop(0, chunks, body, init)

    s = lax.rev(jnp.cumsum(lax.rev(chunk_sav, (0,)), axis=0), (0,))
    cond = (acc_above + s) >= kt
    lane = jnp.sum(cond.astype(jnp.int32)) - 1
    hist_lane = jnp.sum(jnp.where(lane_iota == lane, chunk_sav, 0))
    s_lane = jnp.sum(jnp.where(lane_iota == lane, s, 0))
    above = acc_above + s_lane - hist_lane
    bi = b_chunk * L + lane
    return bi, kt - above


def _zero_hist(hist_ref, nbins):
    zeros = jnp.zeros((L,), jnp.int32)

    def body(i, _):
        hist_ref[pl.ds(i * L, L)] = zeros
        return jnp.int32(0)

    lax.fori_loop(0, nbins // L, body, jnp.int32(0))


def _process_row(row_v, hist_v):
    ones = jnp.ones((L,), jnp.int32)

    _zero_hist(hist_v, 2048)

    def h1(i, _):
        v = row_v[pl.ds(i * L, L)]
        mu = lax.bitcast_convert_type(_mono(v), jnp.uint32)
        bin1 = jnp.bitwise_xor(jnp.right_shift(mu, jnp.uint32(21)),
                               jnp.uint32(1024)).astype(jnp.int32)
        plsc.addupdate_scatter(hist_v, [bin1], ones)
        return jnp.int32(0)

    lax.fori_loop(0, NCHUNK, h1, jnp.int32(0))
    b1, k2 = _find_bin(hist_v, 2048, jnp.int32(K))
    t11k = jnp.bitwise_xor(b1, 1024)
    t11k_u = t11k.astype(jnp.uint32)

    _zero_hist(hist_v, 2048)

    def h2(i, _):
        v = row_v[pl.ds(i * L, L)]
        mu = lax.bitcast_convert_type(_mono(v), jnp.uint32)
        sel = jnp.right_shift(mu, jnp.uint32(21)) == t11k_u
        bin2 = jnp.bitwise_and(jnp.right_shift(mu, jnp.uint32(10)),
                               jnp.uint32(0x7FF)).astype(jnp.int32)
        plsc.addupdate_scatter(hist_v, [bin2], ones, mask=sel)
        return jnp.int32(0)

    lax.fori_loop(0, NCHUNK, h2, jnp.int32(0))
    b2, k3 = _find_bin(hist_v, 2048, k2)
    top22k_u = jnp.bitwise_or(
        jnp.left_shift(t11k_u, jnp.uint32(11)), b2.astype(jnp.uint32))

    _zero_hist(hist_v, 1024)

    def h3(i, _):
        v = row_v[pl.ds(i * L, L)]
        mu = lax.bitcast_convert_type(_mono(v), jnp.uint32)
        sel = jnp.right_shift(mu, jnp.uint32(10)) == top22k_u
        bin3 = jnp.bitwise_and(mu, jnp.uint32(0x3FF)).astype(jnp.int32)
        plsc.addupdate_scatter(hist_v, [bin3], ones, mask=sel)
        return jnp.int32(0)

    lax.fori_loop(0, NCHUNK, h3, jnp.int32(0))
    b3, k4 = _find_bin(hist_v, 1024, k3)

    mk = jnp.bitwise_or(
        jnp.bitwise_or(jnp.left_shift(t11k, 21), jnp.left_shift(b2, 10)), b3)
    need = k4

    def mask_body(i, cnt):
        sl = pl.ds(i * L, L)
        v = row_v[sl]
        m = _mono(v)
        eq = m == mk
        pc = jnp.cumsum(eq.astype(jnp.int32))
        keep = jnp.logical_or(m > mk,
                              jnp.logical_and(eq, (cnt + pc) <= need))
        row_v[sl] = jnp.where(keep, v, jnp.float32(0))
        return cnt + jnp.sum(eq.astype(jnp.int32))

    lax.fori_loop(0, NCHUNK, mask_body, jnp.int32(0))


@functools.partial(
    pl.kernel,
    out_type=jax.ShapeDtypeStruct((B * N,), jnp.float32),
    mesh=plsc.VectorSubcoreMesh(core_axis_name="c", subcore_axis_name="s"),
    scratch_types=[
        pltpu.VMEM((N,), jnp.float32),
        pltpu.VMEM((2048,), jnp.int32),
    ],
    compiler_params=pltpu.CompilerParams(needs_layout_passes=False),
)
def _topk_mask_sc(x_hbm, out_hbm, row_v, hist_v):
    wid = lax.axis_index("s") * 2 + lax.axis_index("c")

    def row_body(r, _):
        row = wid * ROWS_PER_W + r
        base = row * N
        pltpu.sync_copy(x_hbm.at[pl.ds(base, N)], row_v)
        _process_row(row_v, hist_v)
        pltpu.sync_copy(row_v, out_hbm.at[pl.ds(base, N)])
        return jnp.int32(0)

    lax.fori_loop(0, ROWS_PER_W, row_body, jnp.int32(0))


@jax.jit
def kernel(x):
    return _topk_mask_sc(x.reshape(-1)).reshape(x.shape)


# speedup vs baseline: 11.7800x; 1.4108x over previous
import functools

import jax
import jax.numpy as jnp
from jax import lax
from jax.experimental import pallas as pl
from jax.experimental.pallas import tpu as pltpu
from jax.experimental.pallas import tpu_sc as plsc

B = 128
N = 32768
K = N // 8
L = 16
NCHUNK = N // L
NW = 32
ROWS_PER_W = B // NW
UNROLL = 8

_MASK31 = 0x7FFFFFFF


def _mono(v):
    b = lax.bitcast_convert_type(v, jnp.int32)
    s = jnp.right_shift(b, 31)
    m = jnp.bitwise_xor(b, jnp.bitwise_and(s, _MASK31))
    return lax.bitcast_convert_type(m, jnp.uint32)


def _mono_i32(v):
    b = lax.bitcast_convert_type(v, jnp.int32)
    s = jnp.right_shift(b, 31)
    return jnp.bitwise_xor(b, jnp.bitwise_and(s, _MASK31))


def _find_bin(hist_ref, nbins, kt):
    chunks = nbins // L
    lane_iota = lax.iota(jnp.int32, L)
    cu = 4

    def body(i, carry):
        acc, b_chunk, acc_above, chunk_sav = carry
        for u in range(cu):
            j = chunks - 1 - (i * cu + u)
            chunk = hist_ref[pl.ds(j * L, L)]
            csum = jnp.sum(chunk)
            take = jnp.logical_and(b_chunk < 0, acc + csum >= kt)
            b_chunk = jnp.where(take, j, b_chunk).astype(jnp.int32)
            acc_above = jnp.where(take, acc, acc_above)
            takev = jnp.broadcast_to(take, (L,))
            chunk_sav = jnp.where(takev, chunk, chunk_sav)
            acc = acc + csum
        return acc, b_chunk, acc_above, chunk_sav

    init = (jnp.int32(0), jnp.int32(-1), jnp.int32(0), jnp.zeros((L,), jnp.int32))
    acc, b_chunk, acc_above, chunk_sav = lax.fori_loop(0, chunks // cu, body, init)

    s = lax.rev(jnp.cumsum(lax.rev(chunk_sav, (0,)), axis=0), (0,))
    cond = (acc_above + s) >= kt
    lane = jnp.sum(cond.astype(jnp.int32)) - 1
    hist_lane = jnp.sum(jnp.where(lane_iota == lane, chunk_sav, 0))
    s_lane = jnp.sum(jnp.where(lane_iota == lane, s, 0))
    above = acc_above + s_lane - hist_lane
    bi = b_chunk * L + lane
    return bi, kt - above, hist_lane


def _zero_hist(hist_ref, nbins):
    zeros = jnp.zeros((L,), jnp.int32)

    def body(i, _):
        for u in range(UNROLL):
            hist_ref[pl.ds((i * UNROLL + u) * L, L)] = zeros
        return jnp.int32(0)

    lax.fori_loop(0, nbins // L // UNROLL, body, jnp.int32(0))


def _process_row(row_v, hist_v):
    ones = jnp.ones((L,), jnp.int32)

    _zero_hist(hist_v, 2048)

    def h1(i, _):
        for u in range(UNROLL):
            v = row_v[pl.ds((i * UNROLL + u) * L, L)]
            mu = _mono(v)
            bin1 = lax.bitcast_convert_type(
                jnp.bitwise_xor(jnp.right_shift(mu, jnp.uint32(21)),
                                jnp.uint32(1024)), jnp.int32)
            plsc.addupdate_scatter(hist_v, [bin1], ones)
        return jnp.int32(0)

    lax.fori_loop(0, NCHUNK // UNROLL, h1, jnp.int32(0))
    b1, k2, _ = _find_bin(hist_v, 2048, jnp.int32(K))
    t11k = jnp.bitwise_xor(b1, 1024)
    t11k_u = t11k.astype(jnp.uint32)

    _zero_hist(hist_v, 2048)

    def h2(i, _):
        for u in range(UNROLL):
            v = row_v[pl.ds((i * UNROLL + u) * L, L)]
            mu = _mono(v)
            sel = jnp.right_shift(mu, jnp.uint32(21)) == t11k_u
            bin2 = lax.bitcast_convert_type(
                jnp.bitwise_and(jnp.right_shift(mu, jnp.uint32(10)),
                                jnp.uint32(0x7FF)), jnp.int32)
            plsc.addupdate_scatter(hist_v, [bin2], ones, mask=sel)
        return jnp.int32(0)

    lax.fori_loop(0, NCHUNK // UNROLL, h2, jnp.int32(0))
    b2, k3, _ = _find_bin(hist_v, 2048, k2)
    top22k_u = jnp.bitwise_or(
        jnp.left_shift(t11k_u, jnp.uint32(11)), b2.astype(jnp.uint32))

    _zero_hist(hist_v, 1024)

    def h3(i, _):
        for u in range(UNROLL):
            v = row_v[pl.ds((i * UNROLL + u) * L, L)]
            mu = _mono(v)
            sel = jnp.right_shift(mu, jnp.uint32(10)) == top22k_u
            bin3 = lax.bitcast_convert_type(
                jnp.bitwise_and(mu, jnp.uint32(0x3FF)), jnp.int32)
            plsc.addupdate_scatter(hist_v, [bin3], ones, mask=sel)
        return jnp.int32(0)

    lax.fori_loop(0, NCHUNK // UNROLL, h3, jnp.int32(0))
    b3, k4, hist3 = _find_bin(hist_v, 1024, k3)

    mk = jnp.bitwise_or(
        jnp.bitwise_or(jnp.left_shift(t11k, 21), jnp.left_shift(b2, 10)), b3)

    def mask_body(i, _):
        for u in range(UNROLL):
            sl = pl.ds((i * UNROLL + u) * L, L)
            v = row_v[sl]
            m = _mono_i32(v)
            row_v[sl] = jnp.where(m >= mk, v, jnp.float32(0))
        return jnp.int32(0)

    lax.fori_loop(0, NCHUNK // UNROLL, mask_body, jnp.int32(0))

    extra = hist3 - k4

    def fix_cond(carry):
        j, ex = carry
        return jnp.logical_and(ex > 0, j >= 0)

    def fix_body(carry):
        j, ex = carry
        sl = pl.ds(j * L, L)
        v = row_v[sl]
        eq = _mono_i32(v) == mk
        eqi = eq.astype(jnp.int32)
        rpc = lax.rev(jnp.cumsum(lax.rev(eqi, (0,)), axis=0), (0,))
        drop = jnp.logical_and(eq, rpc <= ex)
        row_v[sl] = jnp.where(drop, jnp.float32(0), v)
        ncnt = jnp.sum(eqi)
        ex = jnp.maximum(ex - ncnt, 0)
        return j - 1, ex

    lax.while_loop(fix_cond, fix_body, (jnp.int32(NCHUNK - 1), extra))


@functools.partial(
    pl.kernel,
    out_type=jax.ShapeDtypeStruct((B * N,), jnp.float32),
    mesh=plsc.VectorSubcoreMesh(core_axis_name="c", subcore_axis_name="s"),
    scratch_types=[
        pltpu.VMEM((N,), jnp.float32),
        pltpu.VMEM((2048,), jnp.int32),
    ],
    compiler_params=pltpu.CompilerParams(needs_layout_passes=False),
)
def _topk_mask_sc(x_hbm, out_hbm, row_v, hist_v):
    wid = lax.axis_index("s") * 2 + lax.axis_index("c")

    def row_body(r, _):
        row = wid * ROWS_PER_W + r
        base = row * N
        pltpu.sync_copy(x_hbm.at[pl.ds(base, N)], row_v)
        _process_row(row_v, hist_v)
        pltpu.sync_copy(row_v, out_hbm.at[pl.ds(base, N)])
        return jnp.int32(0)

    lax.fori_loop(0, ROWS_PER_W, row_body, jnp.int32(0))


@jax.jit
def kernel(x):
    return _topk_mask_sc(x.reshape(-1)).reshape(x.shape)


# speedup vs baseline: 32.3534x; 2.7465x over previous
import functools

import jax
import jax.numpy as jnp
from jax import lax
from jax.experimental import pallas as pl
from jax.experimental.pallas import tpu as pltpu
from jax.experimental.pallas import tpu_sc as plsc

B = 128
N = 32768
K = N // 8
L = 16
NCHUNK = N // L
NW = 32
ROWS_PER_W = B // NW
UNROLL = 8

_MASK31 = 0x7FFFFFFF


def _mono(v):
    b = lax.bitcast_convert_type(v, jnp.int32)
    s = jnp.right_shift(b, 31)
    m = jnp.bitwise_xor(b, jnp.bitwise_and(s, _MASK31))
    return lax.bitcast_convert_type(m, jnp.uint32)


def _mono_i32(v):
    b = lax.bitcast_convert_type(v, jnp.int32)
    s = jnp.right_shift(b, 31)
    return jnp.bitwise_xor(b, jnp.bitwise_and(s, _MASK31))


def _find_bin(hist_ref, nbins, kt):
    chunks = nbins // L
    lane_iota = lax.iota(jnp.int32, L)
    init = (jnp.int32(0), jnp.int32(-1), jnp.int32(0), jnp.zeros((L,), jnp.int32))

    def body(i, carry):
        acc, b_chunk, acc_above, chunk_sav = carry
        j = chunks - 1 - i
        chunk = hist_ref[pl.ds(j * L, L)]
        csum = jnp.sum(chunk)
        take = jnp.logical_and(b_chunk < 0, acc + csum >= kt)
        b_chunk = jnp.where(take, j, b_chunk).astype(jnp.int32)
        acc_above = jnp.where(take, acc, acc_above)
        takev = jnp.broadcast_to(take, (L,))
        chunk_sav = jnp.where(takev, chunk, chunk_sav)
        return acc + csum, b_chunk, acc_above, chunk_sav

    acc, b_chunk, acc_above, chunk_sav = plsc.parallel_loop(
        0, chunks, 1, unroll=4, carry=init)(body)

    s = lax.rev(jnp.cumsum(lax.rev(chunk_sav, (0,)), axis=0), (0,))
    cond = (acc_above + s) >= kt
    lane = jnp.sum(cond.astype(jnp.int32)) - 1
    hist_lane = jnp.sum(jnp.where(lane_iota == lane, chunk_sav, 0))
    s_lane = jnp.sum(jnp.where(lane_iota == lane, s, 0))
    above = acc_above + s_lane - hist_lane
    bi = b_chunk * L + lane
    return bi, kt - above, hist_lane


def _zero_hist(hist_ref, nbins):
    zeros = jnp.zeros((L,), jnp.int32)

    @plsc.parallel_loop(0, nbins // L, 1, unroll=UNROLL)
    def _(i):
        hist_ref[pl.ds(i * L, L)] = zeros


def _process_row(row_v, hist_v):
    ones = jnp.ones((L,), jnp.int32)

    _zero_hist(hist_v, 2048)

    @plsc.parallel_loop(0, NCHUNK, 1, unroll=UNROLL)
    def _(i):
        v = row_v[pl.ds(i * L, L)]
        mu = _mono(v)
        bin1 = lax.bitcast_convert_type(
            jnp.bitwise_xor(jnp.right_shift(mu, jnp.uint32(21)),
                            jnp.uint32(1024)), jnp.int32)
        plsc.addupdate_scatter(hist_v, [bin1], ones)

    b1, k2, _ = _find_bin(hist_v, 2048, jnp.int32(K))
    t11k = jnp.bitwise_xor(b1, 1024)
    t11k_u = t11k.astype(jnp.uint32)

    _zero_hist(hist_v, 2048)

    @plsc.parallel_loop(0, NCHUNK, 1, unroll=UNROLL)
    def _(i):
        v = row_v[pl.ds(i * L, L)]
        mu = _mono(v)
        sel = jnp.right_shift(mu, jnp.uint32(21)) == t11k_u
        bin2 = lax.bitcast_convert_type(
            jnp.bitwise_and(jnp.right_shift(mu, jnp.uint32(10)),
                            jnp.uint32(0x7FF)), jnp.int32)
        plsc.addupdate_scatter(hist_v, [bin2], ones, mask=sel)

    b2, k3, _ = _find_bin(hist_v, 2048, k2)
    top22k_u = jnp.bitwise_or(
        jnp.left_shift(t11k_u, jnp.uint32(11)), b2.astype(jnp.uint32))

    _zero_hist(hist_v, 1024)

    @plsc.parallel_loop(0, NCHUNK, 1, unroll=UNROLL)
    def _(i):
        v = row_v[pl.ds(i * L, L)]
        mu = _mono(v)
        sel = jnp.right_shift(mu, jnp.uint32(10)) == top22k_u
        bin3 = lax.bitcast_convert_type(
            jnp.bitwise_and(mu, jnp.uint32(0x3FF)), jnp.int32)
        plsc.addupdate_scatter(hist_v, [bin3], ones, mask=sel)

    b3, k4, hist3 = _find_bin(hist_v, 1024, k3)

    mk = jnp.bitwise_or(
        jnp.bitwise_or(jnp.left_shift(t11k, 21), jnp.left_shift(b2, 10)), b3)

    @plsc.parallel_loop(0, NCHUNK, 1, unroll=UNROLL)
    def _(i):
        sl = pl.ds(i * L, L)
        v = row_v[sl]
        m = _mono_i32(v)
        row_v[sl] = jnp.where(m >= mk, v, jnp.float32(0))

    extra = hist3 - k4

    def fix_cond(carry):
        j, ex = carry
        return jnp.logical_and(ex > 0, j >= 0)

    def fix_body(carry):
        j, ex = carry
        sl = pl.ds(j * L, L)
        v = row_v[sl]
        eq = _mono_i32(v) == mk
        eqi = eq.astype(jnp.int32)
        rpc = lax.rev(jnp.cumsum(lax.rev(eqi, (0,)), axis=0), (0,))
        drop = jnp.logical_and(eq, rpc <= ex)
        row_v[sl] = jnp.where(drop, jnp.float32(0), v)
        ncnt = jnp.sum(eqi)
        ex = jnp.maximum(ex - ncnt, 0)
        return j - 1, ex

    lax.while_loop(fix_cond, fix_body, (jnp.int32(NCHUNK - 1), extra))


@functools.partial(
    pl.kernel,
    out_type=jax.ShapeDtypeStruct((B * N,), jnp.float32),
    mesh=plsc.VectorSubcoreMesh(core_axis_name="c", subcore_axis_name="s"),
    scratch_types=[
        pltpu.VMEM((N,), jnp.float32),
        pltpu.VMEM((2048,), jnp.int32),
    ],
    compiler_params=pltpu.CompilerParams(needs_layout_passes=False),
)
def _topk_mask_sc(x_hbm, out_hbm, row_v, hist_v):
    wid = lax.axis_index("s") * 2 + lax.axis_index("c")

    def row_body(r, _):
        row = wid * ROWS_PER_W + r
        base = row * N
        pltpu.sync_copy(x_hbm.at[pl.ds(base, N)], row_v)
        _process_row(row_v, hist_v)
        pltpu.sync_copy(row_v, out_hbm.at[pl.ds(base, N)])
        return jnp.int32(0)

    lax.fori_loop(0, ROWS_PER_W, row_body, jnp.int32(0))


@jax.jit
def kernel(x):
    return _topk_mask_sc(x.reshape(-1)).reshape(x.shape)


# speedup vs baseline: 32.6907x; 1.0104x over previous
import functools

import jax
import jax.numpy as jnp
from jax import lax
from jax.experimental import pallas as pl
from jax.experimental.pallas import tpu as pltpu
from jax.experimental.pallas import tpu_sc as plsc

B = 128
N = 32768
K = N // 8
L = 16
NCHUNK = N // L
NW = 32
ROWS_PER_W = B // NW
UNROLL = 8

_MASK31 = 0x7FFFFFFF


def _mono(v):
    b = lax.bitcast_convert_type(v, jnp.int32)
    s = jnp.right_shift(b, 31)
    m = jnp.bitwise_xor(b, jnp.bitwise_and(s, _MASK31))
    return lax.bitcast_convert_type(m, jnp.uint32)


def _mono_i32(v):
    b = lax.bitcast_convert_type(v, jnp.int32)
    s = jnp.right_shift(b, 31)
    return jnp.bitwise_xor(b, jnp.bitwise_and(s, _MASK31))


def _find_bin(hist_ref, nbins, kt):
    chunks = nbins // L
    lane_iota = lax.iota(jnp.int32, L)
    init = (jnp.int32(0), jnp.int32(-1), jnp.int32(0), jnp.zeros((L,), jnp.int32))

    def body(i, carry):
        acc, b_chunk, acc_above, chunk_sav = carry
        j = chunks - 1 - i
        chunk = hist_ref[pl.ds(j * L, L)]
        csum = jnp.sum(chunk)
        take = jnp.logical_and(b_chunk < 0, acc + csum >= kt)
        b_chunk = jnp.where(take, j, b_chunk).astype(jnp.int32)
        acc_above = jnp.where(take, acc, acc_above)
        takev = jnp.broadcast_to(take, (L,))
        chunk_sav = jnp.where(takev, chunk, chunk_sav)
        return acc + csum, b_chunk, acc_above, chunk_sav

    acc, b_chunk, acc_above, chunk_sav = plsc.parallel_loop(
        0, chunks, 1, unroll=4, carry=init)(body)

    s = lax.rev(jnp.cumsum(lax.rev(chunk_sav, (0,)), axis=0), (0,))
    cond = (acc_above + s) >= kt
    lane = jnp.sum(cond.astype(jnp.int32)) - 1
    hist_lane = jnp.sum(jnp.where(lane_iota == lane, chunk_sav, 0))
    s_lane = jnp.sum(jnp.where(lane_iota == lane, s, 0))
    above = acc_above + s_lane - hist_lane
    bi = b_chunk * L + lane
    return bi, kt - above, hist_lane


def _zero_hist(hist_ref, nbins):
    zeros = jnp.zeros((L,), jnp.int32)

    @plsc.parallel_loop(0, nbins // L, 1, unroll=UNROLL)
    def _(i):
        hist_ref[pl.ds(i * L, L)] = zeros


def _process_row(row_v, hist_v):
    ones = jnp.ones((L,), jnp.int32)

    _zero_hist(hist_v, 2048)

    @plsc.parallel_loop(0, NCHUNK, 1, unroll=UNROLL)
    def _(i):
        v = row_v[pl.ds(i * L, L)]
        mu = _mono(v)
        bin1 = lax.bitcast_convert_type(
            jnp.bitwise_xor(jnp.right_shift(mu, jnp.uint32(21)),
                            jnp.uint32(1024)), jnp.int32)
        plsc.addupdate_scatter(hist_v, [bin1], ones)

    b1, k2, _ = _find_bin(hist_v, 2048, jnp.int32(K))
    t11k = jnp.bitwise_xor(b1, 1024)
    t11k_u = t11k.astype(jnp.uint32)

    _zero_hist(hist_v, 2048)

    @plsc.parallel_loop(0, NCHUNK, 1, unroll=UNROLL)
    def _(i):
        v = row_v[pl.ds(i * L, L)]
        mu = _mono(v)
        sel = jnp.right_shift(mu, jnp.uint32(21)) == t11k_u
        bin2 = lax.bitcast_convert_type(
            jnp.bitwise_and(jnp.right_shift(mu, jnp.uint32(10)),
                            jnp.uint32(0x7FF)), jnp.int32)
        plsc.addupdate_scatter(hist_v, [bin2], ones, mask=sel)

    b2, k3, _ = _find_bin(hist_v, 2048, k2)
    top22k_u = jnp.bitwise_or(
        jnp.left_shift(t11k_u, jnp.uint32(11)), b2.astype(jnp.uint32))

    _zero_hist(hist_v, 1024)

    @plsc.parallel_loop(0, NCHUNK, 1, unroll=UNROLL)
    def _(i):
        v = row_v[pl.ds(i * L, L)]
        mu = _mono(v)
        sel = jnp.right_shift(mu, jnp.uint32(10)) == top22k_u
        bin3 = lax.bitcast_convert_type(
            jnp.bitwise_and(mu, jnp.uint32(0x3FF)), jnp.int32)
        plsc.addupdate_scatter(hist_v, [bin3], ones, mask=sel)

    b3, k4, hist3 = _find_bin(hist_v, 1024, k3)

    mk = jnp.bitwise_or(
        jnp.bitwise_or(jnp.left_shift(t11k, 21), jnp.left_shift(b2, 10)), b3)

    @plsc.parallel_loop(0, NCHUNK, 1, unroll=UNROLL)
    def _(i):
        sl = pl.ds(i * L, L)
        v = row_v[sl]
        m = _mono_i32(v)
        row_v[sl] = jnp.where(m >= mk, v, jnp.float32(0))

    extra = hist3 - k4

    def fix_cond(carry):
        j, ex = carry
        return jnp.logical_and(ex > 0, j >= 0)

    def fix_body(carry):
        j, ex = carry
        sl = pl.ds(j * L, L)
        v = row_v[sl]
        eq = _mono_i32(v) == mk
        eqi = eq.astype(jnp.int32)
        rpc = lax.rev(jnp.cumsum(lax.rev(eqi, (0,)), axis=0), (0,))
        drop = jnp.logical_and(eq, rpc <= ex)
        row_v[sl] = jnp.where(drop, jnp.float32(0), v)
        ncnt = jnp.sum(eqi)
        ex = jnp.maximum(ex - ncnt, 0)
        return j - 1, ex

    lax.while_loop(fix_cond, fix_body, (jnp.int32(NCHUNK - 1), extra))


@functools.partial(
    pl.kernel,
    out_type=jax.ShapeDtypeStruct((B * N,), jnp.float32),
    mesh=plsc.VectorSubcoreMesh(core_axis_name="c", subcore_axis_name="s"),
    scratch_types=[
        pltpu.VMEM((N,), jnp.float32),
        pltpu.VMEM((N,), jnp.float32),
        pltpu.VMEM((2048,), jnp.int32),
        pltpu.SemaphoreType.DMA,
        pltpu.SemaphoreType.DMA,
        pltpu.SemaphoreType.DMA,
        pltpu.SemaphoreType.DMA,
    ],
    compiler_params=pltpu.CompilerParams(needs_layout_passes=False),
)
def _topk_mask_sc(x_hbm, out_hbm, row_a, row_b, hist_v,
                  sem_ia, sem_ib, sem_oa, sem_ob):
    wid = lax.axis_index("s") * 2 + lax.axis_index("c")
    bufs = (row_a, row_b)
    isems = (sem_ia, sem_ib)
    osems = (sem_oa, sem_ob)

    def _in_slice(r):
        return x_hbm.at[pl.ds((wid * ROWS_PER_W + r) * N, N)]

    def _out_slice(r):
        return out_hbm.at[pl.ds((wid * ROWS_PER_W + r) * N, N)]

    for r in range(2):
        pltpu.async_copy(_in_slice(r), bufs[r], isems[r])

    for r in range(ROWS_PER_W):
        bi = r % 2
        pltpu.make_async_copy(_in_slice(r), bufs[bi], isems[bi]).wait()
        _process_row(bufs[bi], hist_v)
        pltpu.async_copy(bufs[bi], _out_slice(r), osems[bi])
        if r + 2 < ROWS_PER_W:
            pltpu.make_async_copy(bufs[bi], _out_slice(r), osems[bi]).wait()
            pltpu.async_copy(_in_slice(r + 2), bufs[bi], isems[bi])

    for r in (ROWS_PER_W - 2, ROWS_PER_W - 1):
        bi = r % 2
        pltpu.make_async_copy(bufs[bi], _out_slice(r), osems[bi]).wait()


@jax.jit
def kernel(x):
    return _topk_mask_sc(x.reshape(-1)).reshape(x.shape)
